# baseline (speedup 1.0000x reference)
import jax
import jax.numpy as jnp
from jax.experimental import pallas as pl


def _fused_kernel(gu_ref, gi_ref, xui_ref, gu_out_ref, gi_out_ref):
    u = gu_ref[:]
    v = gi_ref[:]
    gu_out_ref[:] = u
    gi_out_ref[:] = v
    xui_ref[:] = jnp.sum(u * v, axis=1)


def kernel(gu, gi):
    B, D = gu.shape
    BLK = 1024
    xui, gu_out, gi_out = pl.pallas_call(
        _fused_kernel,
        grid=(B // BLK,),
        in_specs=[
            pl.BlockSpec((BLK, D), lambda i: (i, 0)),
            pl.BlockSpec((BLK, D), lambda i: (i, 0)),
        ],
        out_specs=[
            pl.BlockSpec((BLK,), lambda i: (i,)),
            pl.BlockSpec((BLK, D), lambda i: (i, 0)),
            pl.BlockSpec((BLK, D), lambda i: (i, 0)),
        ],
        out_shape=[
            jax.ShapeDtypeStruct((B,), jnp.float32),
            jax.ShapeDtypeStruct((B, D), jnp.float32),
            jax.ShapeDtypeStruct((B, D), jnp.float32),
        ],
    )(gu, gi)
    return (xui, gu_out, gi_out)
